# baseline (speedup 1.0000x reference)
import functools
import math

import jax
import jax.numpy as jnp
from jax import lax
from jax.experimental import pallas as pl
from jax.experimental.pallas import tpu as pltpu
from jax.experimental.pallas import tpu_sc as plsc

B = 4096
NCHUNK = 2
BCH = B // NCHUNK
T = 100
D = 128
NC = 2
NS = 16
NW = NC * NS
BC = BCH // NW
TP = T
NBUF = 4
WBUF = 2
SCALE = math.sqrt(D)
L = 16
VPR = D // L

_mesh = plsc.VectorSubcoreMesh(core_axis_name="c", subcore_axis_name="s")


@functools.partial(
    pl.kernel,
    mesh=_mesh,
    out_type=jax.ShapeDtypeStruct((BCH, T, D), jnp.float32),
    scratch_types=[
        pltpu.VMEM((BC, TP), jnp.int32),
        pltpu.VMEM((T, D), jnp.float32),
        pltpu.VMEM((NBUF, T, D), jnp.float32),
        pltpu.VMEM((WBUF, T, D), jnp.float32),
        pltpu.SemaphoreType.DMA((NBUF,)),
        pltpu.SemaphoreType.DMA((WBUF,)),
    ],
)
def _emb_lookup(inp_hbm, table_hbm, pos_hbm, out_hbm,
                idx_v, pos_v, rows_v, outb_v, gsem, wsem):
    w = lax.axis_index("s") * NC + lax.axis_index("c")
    b0 = w * BC

    pltpu.sync_copy(inp_hbm.at[pl.ds(b0, BC)], idx_v)
    pltpu.sync_copy(pos_hbm, pos_v)

    def idx_list(q):
        return idx_v.at[q, pl.ds(0, T)]

    for k in range(NBUF):
        pltpu.async_copy(table_hbm.at[idx_list(k)], rows_v.at[k], gsem.at[k])

    def outer(i, carry):
        for k in range(NBUF):
            q = i * NBUF + k
            b = b0 + q
            kw = k % WBUF
            pltpu.make_async_copy(
                table_hbm.at[idx_list(q)], rows_v.at[k], gsem.at[k]).wait()

            def wait_write():
                pltpu.make_async_copy(
                    outb_v.at[kw], out_hbm.at[b], wsem.at[kw]).wait()

            if k < WBUF:
                pl.when(i > 0)(wait_write)
            else:
                wait_write()

            def row_body(r, c):
                for j in range(VPR):
                    sl = pl.ds(j * L, L)
                    outb_v[kw, r, sl] = rows_v[k, r, sl] * SCALE + pos_v[r, sl]
                return c

            lax.fori_loop(0, T, row_body, 0, unroll=2)

            pltpu.async_copy(outb_v.at[kw], out_hbm.at[b], wsem.at[kw])

            @pl.when(q + NBUF < BC)
            def _():
                pltpu.async_copy(
                    table_hbm.at[idx_list(q + NBUF)], rows_v.at[k], gsem.at[k])
        return carry

    lax.fori_loop(0, BC // NBUF, outer, 0)

    for k in range(WBUF):
        b_last = b0 + BC - WBUF + k
        pltpu.make_async_copy(
            outb_v.at[k], out_hbm.at[b_last], wsem.at[k]).wait()


def kernel(inputs, table, pos_table):
    chunks = [
        _emb_lookup(inputs[c * BCH:(c + 1) * BCH], table, pos_table)
        for c in range(NCHUNK)
    ]
    return jnp.concatenate(chunks, axis=0)


# speedup vs baseline: 1.2172x; 1.2172x over previous
import functools
import math

import jax
import jax.numpy as jnp
from jax import lax
from jax.experimental import pallas as pl
from jax.experimental.pallas import tpu as pltpu
from jax.experimental.pallas import tpu_sc as plsc

B = 4096
NCHUNK = 1
BCH = B // NCHUNK
T = 100
D = 128
NC = 2
NS = 16
NW = NC * NS
BC = BCH // NW
TP = T
NBUF = 4
WBUF = 2
SCALE = math.sqrt(D)
L = 16
VPR = D // L

_mesh = plsc.VectorSubcoreMesh(core_axis_name="c", subcore_axis_name="s")


@functools.partial(
    pl.kernel,
    mesh=_mesh,
    out_type=jax.ShapeDtypeStruct((BCH, T, D), jnp.float32),
    scratch_types=[
        pltpu.VMEM((BC, TP), jnp.int32),
        pltpu.VMEM((T, D), jnp.float32),
        pltpu.VMEM((NBUF, T, D), jnp.float32),
        pltpu.VMEM((WBUF, T, D), jnp.float32),
        pltpu.SemaphoreType.DMA((NBUF,)),
        pltpu.SemaphoreType.DMA((WBUF,)),
    ],
)
def _emb_lookup(inp_hbm, table_hbm, pos_hbm, out_hbm,
                idx_v, pos_v, rows_v, outb_v, gsem, wsem):
    w = lax.axis_index("s") * NC + lax.axis_index("c")
    b0 = w * BC

    pltpu.sync_copy(inp_hbm.at[pl.ds(b0, BC)], idx_v)
    pltpu.sync_copy(pos_hbm, pos_v)

    def idx_list(q):
        return idx_v.at[q, pl.ds(0, T)]

    for k in range(NBUF):
        pltpu.async_copy(table_hbm.at[idx_list(k)], rows_v.at[k], gsem.at[k])

    def outer(i, carry):
        for k in range(0, NBUF, 2):
            q = i * NBUF + k
            b = b0 + q
            pltpu.make_async_copy(
                table_hbm.at[idx_list(q)], rows_v.at[k], gsem.at[k]).wait()
            pltpu.make_async_copy(
                table_hbm.at[idx_list(q + 1)], rows_v.at[k + 1],
                gsem.at[k + 1]).wait()

            def wait_writes():
                pltpu.make_async_copy(
                    outb_v.at[0], out_hbm.at[b], wsem.at[0]).wait()
                pltpu.make_async_copy(
                    outb_v.at[1], out_hbm.at[b], wsem.at[1]).wait()

            if k == 0:
                pl.when(i > 0)(wait_writes)
            else:
                wait_writes()

            def row_body(r, c):
                for j in range(VPR):
                    sl = pl.ds(j * L, L)
                    p = pos_v[r, sl]
                    outb_v[0, r, sl] = rows_v[k, r, sl] * SCALE + p
                    outb_v[1, r, sl] = rows_v[k + 1, r, sl] * SCALE + p
                return c

            lax.fori_loop(0, T, row_body, 0, unroll=2)

            pltpu.async_copy(outb_v.at[0], out_hbm.at[b], wsem.at[0])
            pltpu.async_copy(outb_v.at[1], out_hbm.at[b + 1], wsem.at[1])

            @pl.when(q + NBUF < BC)
            def _():
                pltpu.async_copy(
                    table_hbm.at[idx_list(q + NBUF)], rows_v.at[k], gsem.at[k])
                pltpu.async_copy(
                    table_hbm.at[idx_list(q + NBUF + 1)], rows_v.at[k + 1],
                    gsem.at[k + 1])
        return carry

    lax.fori_loop(0, BC // NBUF, outer, 0)

    for k in range(WBUF):
        b_last = b0 + BC - WBUF + k
        pltpu.make_async_copy(
            outb_v.at[k], out_hbm.at[b_last], wsem.at[k]).wait()


def kernel(inputs, table, pos_table):
    return _emb_lookup(inputs, table, pos_table)


# speedup vs baseline: 2.2350x; 1.8361x over previous
import functools
import math

import jax
import jax.numpy as jnp
from jax import lax
from jax.experimental import pallas as pl
from jax.experimental.pallas import tpu as pltpu
from jax.experimental.pallas import tpu_sc as plsc

B = 4096
NCHUNK = 1
BCH = B // NCHUNK
T = 100
D = 128
NC = 2
NS = 16
NW = NC * NS
BC = BCH // NW
TP = T
NBUF = 4
WBUF = 2
SCALE = math.sqrt(D)
L = 16
VPR = D // L

_mesh = plsc.VectorSubcoreMesh(core_axis_name="c", subcore_axis_name="s")


@functools.partial(
    pl.kernel,
    mesh=_mesh,
    out_type=jax.ShapeDtypeStruct((BCH, T, D), jnp.float32),
    scratch_types=[
        pltpu.VMEM((BC, TP), jnp.int32),
        pltpu.VMEM((T, D), jnp.float32),
        pltpu.VMEM((NBUF, T, D), jnp.float32),
        pltpu.VMEM((WBUF, T, D), jnp.float32),
        pltpu.SemaphoreType.DMA((NBUF,)),
        pltpu.SemaphoreType.DMA((WBUF,)),
    ],
)
def _emb_lookup(inp_hbm, table_hbm, pos_hbm, out_hbm,
                idx_v, pos_v, rows_v, outb_v, gsem, wsem):
    w = lax.axis_index("s") * NC + lax.axis_index("c")
    b0 = w * BC

    pltpu.sync_copy(inp_hbm.at[pl.ds(b0, BC)], idx_v)
    pltpu.sync_copy(pos_hbm, pos_v)

    def idx_list(q):
        return idx_v.at[q, pl.ds(0, T)]

    for k in range(NBUF):
        pltpu.async_copy(table_hbm.at[idx_list(k)], rows_v.at[k], gsem.at[k])

    def outer(i, carry):
        for k in range(NBUF):
            q = i * NBUF + k
            b = b0 + q
            kw = k % WBUF
            pltpu.make_async_copy(
                table_hbm.at[idx_list(q)], rows_v.at[k], gsem.at[k]).wait()

            def wait_write():
                pltpu.make_async_copy(
                    outb_v.at[kw], out_hbm.at[b], wsem.at[kw]).wait()

            if k < WBUF:
                pl.when(i > 0)(wait_write)
            else:
                wait_write()

            @plsc.parallel_loop(0, T, step=1, unroll=4)
            def row_body(r):
                for j in range(VPR):
                    sl = pl.ds(j * L, L)
                    outb_v[kw, r, sl] = rows_v[k, r, sl] * SCALE + pos_v[r, sl]

            pltpu.async_copy(outb_v.at[kw], out_hbm.at[b], wsem.at[kw])

            @pl.when(q + NBUF < BC)
            def _():
                pltpu.async_copy(
                    table_hbm.at[idx_list(q + NBUF)], rows_v.at[k], gsem.at[k])
        return carry

    lax.fori_loop(0, BC // NBUF, outer, 0)

    for k in range(WBUF):
        b_last = b0 + BC - WBUF + k
        pltpu.make_async_copy(
            outb_v.at[k], out_hbm.at[b_last], wsem.at[k]).wait()


def kernel(inputs, table, pos_table):
    return _emb_lookup(inputs, table, pos_table)
